# speedup vs baseline: 3.4022x; 3.4022x over previous
import functools

import jax
import jax.numpy as jnp
from jax.experimental import pallas as pl

_KPAD = 128


def _body(p_ref, lt_ref, hl2_ref, lrows_ref, out_ref):
    p = p_ref[...]
    lt = lt_ref[...]
    scores = hl2_ref[...] - jnp.dot(p, lt, preferred_element_type=jnp.float32)
    m = jnp.min(scores, axis=1, keepdims=True)
    cols = jax.lax.broadcasted_iota(jnp.int32, scores.shape, 1)
    idx = jnp.min(jnp.where(scores == m, cols, _KPAD), axis=1, keepdims=True)
    onehot = (cols == idx).astype(jnp.float32)
    out_ref[...] = jnp.dot(onehot, lrows_ref[...],
                           preferred_element_type=jnp.float32)


@functools.partial(jax.jit, static_argnames=("block",))
def _run(params, lattice_points, block=2048):
    n, d = params.shape
    k = lattice_points.shape[0]
    lrows = jnp.zeros((_KPAD, d), jnp.float32).at[:k].set(lattice_points)
    lt = lrows.T
    hl2 = 0.5 * jnp.sum(lrows * lrows, axis=1)
    hl2 = jnp.where(jnp.arange(_KPAD) < k, hl2, jnp.inf)[None, :]
    grid = (n // block,)
    return pl.pallas_call(
        _body,
        grid=grid,
        in_specs=[
            pl.BlockSpec((block, d), lambda i: (i, 0)),
            pl.BlockSpec((d, _KPAD), lambda i: (0, 0)),
            pl.BlockSpec((1, _KPAD), lambda i: (0, 0)),
            pl.BlockSpec((_KPAD, d), lambda i: (0, 0)),
        ],
        out_specs=pl.BlockSpec((block, d), lambda i: (i, 0)),
        out_shape=jax.ShapeDtypeStruct((n, d), jnp.float32),
    )(params, lt, hl2, lrows)


def kernel(params, lattice_points):
    return _run(params, lattice_points)


# speedup vs baseline: 3.6580x; 1.0752x over previous
import functools

import jax
import jax.numpy as jnp
from jax.experimental import pallas as pl

_KPAD = 128


def _body(p_ref, lt_ref, hl2_ref, lrows_ref, out_ref):
    p = p_ref[...]
    lt = lt_ref[...]
    scores = hl2_ref[...] - jnp.dot(p, lt, preferred_element_type=jnp.float32)
    m = jnp.min(scores, axis=1, keepdims=True)
    cols = jax.lax.broadcasted_iota(
        jnp.int32, scores.shape, 1).astype(jnp.float32)
    idx = jnp.min(jnp.where(scores == m, cols, float(_KPAD)), axis=1,
                  keepdims=True)
    onehot = (cols == idx).astype(jnp.float32)
    out_ref[...] = jnp.dot(onehot, lrows_ref[...],
                           preferred_element_type=jnp.float32)


@functools.partial(jax.jit, static_argnames=("block",))
def _run(params, lattice_points, block=2048):
    n, d = params.shape
    k = lattice_points.shape[0]
    lrows = jnp.zeros((_KPAD, d), jnp.float32).at[:k].set(lattice_points)
    lt = lrows.T
    hl2 = 0.5 * jnp.sum(lrows * lrows, axis=1)
    hl2 = jnp.where(jnp.arange(_KPAD) < k, hl2, jnp.inf)[None, :]
    grid = (n // block,)
    return pl.pallas_call(
        _body,
        grid=grid,
        in_specs=[
            pl.BlockSpec((block, d), lambda i: (i, 0)),
            pl.BlockSpec((d, _KPAD), lambda i: (0, 0)),
            pl.BlockSpec((1, _KPAD), lambda i: (0, 0)),
            pl.BlockSpec((_KPAD, d), lambda i: (0, 0)),
        ],
        out_specs=pl.BlockSpec((block, d), lambda i: (i, 0)),
        out_shape=jax.ShapeDtypeStruct((n, d), jnp.float32),
    )(params, lt, hl2, lrows)


def kernel(params, lattice_points):
    return _run(params, lattice_points)


# speedup vs baseline: 3.7632x; 1.0288x over previous
import functools

import jax
import jax.numpy as jnp
from jax.experimental import pallas as pl

_KPAD = 128


def _body(p_ref, lrows_ref, hl2_ref, out_ref):
    p = p_ref[...]
    lrows = lrows_ref[...]
    scoresT = hl2_ref[...] - jax.lax.dot_general(
        lrows, p, (((1,), (1,)), ((), ())),
        preferred_element_type=jnp.float32)
    m = jnp.min(scoresT, axis=0, keepdims=True)
    rows = jax.lax.broadcasted_iota(
        jnp.int32, scoresT.shape, 0).astype(jnp.float32)
    idx = jnp.min(jnp.where(scoresT == m, rows, float(_KPAD)), axis=0,
                  keepdims=True)
    onehotT = (rows == idx).astype(jnp.float32)
    out_ref[...] = jax.lax.dot_general(
        onehotT, lrows, (((0,), (0,)), ((), ())),
        preferred_element_type=jnp.float32)


@functools.partial(jax.jit, static_argnames=("block",))
def _run(params, lattice_points, block=2048):
    n, d = params.shape
    k = lattice_points.shape[0]
    lrows = jnp.zeros((_KPAD, d), jnp.float32).at[:k].set(lattice_points)
    hl2 = 0.5 * jnp.sum(lrows * lrows, axis=1)
    hl2 = jnp.where(jnp.arange(_KPAD) < k, hl2, jnp.inf)[:, None]
    grid = (n // block,)
    return pl.pallas_call(
        _body,
        grid=grid,
        in_specs=[
            pl.BlockSpec((block, d), lambda i: (i, 0)),
            pl.BlockSpec((_KPAD, d), lambda i: (0, 0)),
            pl.BlockSpec((_KPAD, 1), lambda i: (0, 0)),
        ],
        out_specs=pl.BlockSpec((block, d), lambda i: (i, 0)),
        out_shape=jax.ShapeDtypeStruct((n, d), jnp.float32),
    )(params, lrows, hl2)


def kernel(params, lattice_points):
    return _run(params, lattice_points)


# speedup vs baseline: 4.7290x; 1.2566x over previous
import functools

import jax
import jax.numpy as jnp
from jax.experimental import pallas as pl

_KPAD = 128


def _body(p_ref, lrows_ref, hl2_ref, out_ref):
    p = p_ref[...]
    lrows = lrows_ref[...]
    scoresT = hl2_ref[...] - jax.lax.dot_general(
        lrows, p, (((1,), (1,)), ((), ())),
        preferred_element_type=jnp.float32)
    m = jnp.min(scoresT, axis=0, keepdims=True)
    rows = jax.lax.broadcasted_iota(
        jnp.int32, scoresT.shape, 0).astype(jnp.float32)
    idx = jnp.min(jnp.where(scoresT == m, rows, float(_KPAD)), axis=0,
                  keepdims=True)
    onehotT = (rows == idx).astype(jnp.float32)
    out_ref[...] = jax.lax.dot_general(
        onehotT, lrows, (((0,), (0,)), ((), ())),
        preferred_element_type=jnp.float32)


@functools.partial(jax.jit, static_argnames=("block",))
def _run(params, lattice_points, block=8192):
    n, d = params.shape
    k = lattice_points.shape[0]
    lrows = jnp.zeros((_KPAD, d), jnp.float32).at[:k].set(lattice_points)
    hl2 = 0.5 * jnp.sum(lrows * lrows, axis=1)
    hl2 = jnp.where(jnp.arange(_KPAD) < k, hl2, jnp.inf)[:, None]
    grid = (n // block,)
    return pl.pallas_call(
        _body,
        grid=grid,
        in_specs=[
            pl.BlockSpec((block, d), lambda i: (i, 0)),
            pl.BlockSpec((_KPAD, d), lambda i: (0, 0)),
            pl.BlockSpec((_KPAD, 1), lambda i: (0, 0)),
        ],
        out_specs=pl.BlockSpec((block, d), lambda i: (i, 0)),
        out_shape=jax.ShapeDtypeStruct((n, d), jnp.float32),
    )(params, lrows, hl2)


def kernel(params, lattice_points):
    return _run(params, lattice_points)


# speedup vs baseline: 4.8611x; 1.0279x over previous
import functools

import jax
import jax.numpy as jnp
from jax.experimental import pallas as pl

_KPAD = 128


def _body(p_ref, lrows_ref, hl2_ref, out_ref):
    p = p_ref[...]
    lrows = lrows_ref[...]
    scoresT = hl2_ref[...] - jax.lax.dot_general(
        lrows, p, (((1,), (1,)), ((), ())),
        preferred_element_type=jnp.float32)
    m = jnp.min(scoresT, axis=0, keepdims=True)
    rows = jax.lax.broadcasted_iota(
        jnp.int32, scoresT.shape, 0).astype(jnp.float32)
    idx = jnp.min(jnp.where(scoresT == m, rows, float(_KPAD)), axis=0,
                  keepdims=True)
    onehotT = (rows == idx).astype(jnp.float32)
    out_ref[...] = jax.lax.dot_general(
        onehotT, lrows, (((0,), (0,)), ((), ())),
        preferred_element_type=jnp.float32)


@functools.partial(jax.jit, static_argnames=("block",))
def _run(params, lattice_points, block=16384):
    n, d = params.shape
    k = lattice_points.shape[0]
    lrows = jnp.zeros((_KPAD, d), jnp.float32).at[:k].set(lattice_points)
    hl2 = 0.5 * jnp.sum(lrows * lrows, axis=1)
    hl2 = jnp.where(jnp.arange(_KPAD) < k, hl2, jnp.inf)[:, None]
    grid = (n // block,)
    return pl.pallas_call(
        _body,
        grid=grid,
        in_specs=[
            pl.BlockSpec((block, d), lambda i: (i, 0)),
            pl.BlockSpec((_KPAD, d), lambda i: (0, 0)),
            pl.BlockSpec((_KPAD, 1), lambda i: (0, 0)),
        ],
        out_specs=pl.BlockSpec((block, d), lambda i: (i, 0)),
        out_shape=jax.ShapeDtypeStruct((n, d), jnp.float32),
    )(params, lrows, hl2)


def kernel(params, lattice_points):
    return _run(params, lattice_points)
